# baseline (speedup 1.0000x reference)
import functools

import jax
import jax.numpy as jnp
from jax import lax
from jax.experimental import pallas as pl
from jax.experimental.pallas import tpu as pltpu
from jax.experimental.pallas import tpu_sc as plsc

E = 16
H = 1024
I = 768
K = 2
T = 4096
P = T * K
TILE = 256
NT = P // TILE + E
P_PAD = NT * TILE

NC, NS = 2, 16
NW = NC * NS
TPW = T // NW
CHD = 64
CHC = 32


def _route(top_k_index):
    e = top_k_index.reshape(-1).astype(jnp.int32)
    oh = (e[:, None] == jnp.arange(E, dtype=jnp.int32)[None, :]).astype(jnp.int32)
    cum = jnp.cumsum(oh, axis=0)
    counts = cum[-1]
    rank = jnp.sum(cum * oh, axis=1) - 1
    padded = ((counts + TILE - 1) // TILE) * TILE
    pad_start = jnp.concatenate([jnp.zeros(1, jnp.int32),
                                 jnp.cumsum(padded)[:-1].astype(jnp.int32)])
    pos = (pad_start[e] + rank).reshape(T, K)
    tiles_per_e = padded // TILE
    cum_tiles = jnp.cumsum(tiles_per_e)
    tot = cum_tiles[E - 1].astype(jnp.int32).reshape(1)
    tids = jnp.arange(NT, dtype=jnp.int32)
    eot = jnp.minimum(jnp.searchsorted(cum_tiles, tids, side="right"),
                      E - 1).astype(jnp.int32)
    return pos, eot, tot


@functools.cache
def _dispatch_sc_call():
    mesh = plsc.VectorSubcoreMesh(core_axis_name="c", subcore_axis_name="s")

    @functools.partial(
        pl.kernel,
        mesh=mesh,
        out_type=(jax.ShapeDtypeStruct((P_PAD, H), jnp.float32),
                  jax.ShapeDtypeStruct((P_PAD,), jnp.float32)),
        scratch_types=[
            pltpu.VMEM((CHD, H), jnp.float32),
            pltpu.VMEM((CHD,), jnp.int32),
            pltpu.VMEM((CHD,), jnp.int32),
            pltpu.VMEM((CHD,), jnp.float32),
            pltpu.VMEM((CHD,), jnp.float32),
            pltpu.SemaphoreType.DMA,
        ],
    )
    def dispatch(hidden_hbm, pos_e_hbm, pos_o_hbm, w_e_hbm, w_o_hbm,
                 xpad_hbm, wpad_hbm, rows_v, ie_v, io_v, we_v, wo_v, sem):
        wid = lax.axis_index("s") * NC + lax.axis_index("c")
        _dispatch_body(hidden_hbm, pos_e_hbm, pos_o_hbm, w_e_hbm, w_o_hbm,
                       xpad_hbm, wpad_hbm, rows_v, ie_v, io_v, we_v, wo_v,
                       sem, wid)

    return dispatch


def _dispatch_body(hidden_hbm, pos_e_hbm, pos_o_hbm, w_e_hbm, w_o_hbm,
                   xpad_hbm, wpad_hbm, rows_v, ie_v, io_v, we_v, wo_v,
                   sem, wid):
    for j in range(TPW // CHD):
        base = wid * TPW + j * CHD
        pltpu.sync_copy(pos_e_hbm.at[wid, j], ie_v)
        pltpu.sync_copy(pos_o_hbm.at[wid, j], io_v)
        pltpu.sync_copy(w_e_hbm.at[wid, j], we_v)
        pltpu.sync_copy(w_o_hbm.at[wid, j], wo_v)
        pltpu.sync_copy(hidden_hbm.at[pl.ds(base, CHD)], rows_v)
        c1 = pltpu.async_copy(rows_v, xpad_hbm.at[ie_v], sem)
        c1.wait()
        c2 = pltpu.async_copy(rows_v, xpad_hbm.at[io_v], sem)
        c2.wait()
        c3 = pltpu.async_copy(we_v, wpad_hbm.at[ie_v], sem)
        c3.wait()
        c4 = pltpu.async_copy(wo_v, wpad_hbm.at[io_v], sem)
        c4.wait()


NKC = 3
IC = I // NKC


def _mlp_body(eot_ref, tot_ref, x_ref, wg_ref, wu_ref, wd_ref, wrow_ref,
              y_ref):
    c = pl.program_id(1)

    @pl.when(pl.program_id(0) < tot_ref[0])
    def _():
        x = x_ref[...]
        gate = lax.dot_general(x, wg_ref[0],
                               (((1,), (1,)), ((), ())),
                               preferred_element_type=jnp.float32,
                               precision=lax.Precision.DEFAULT)
        up = lax.dot_general(x, wu_ref[0],
                             (((1,), (1,)), ((), ())),
                             preferred_element_type=jnp.float32,
                             precision=lax.Precision.DEFAULT)
        h = gate * jax.nn.sigmoid(gate) * up * wrow_ref[...]
        y = lax.dot_general(h, wd_ref[0],
                            (((1,), (1,)), ((), ())),
                            preferred_element_type=jnp.float32,
                            precision=lax.Precision.DEFAULT)

        @pl.when(c == 0)
        def _():
            y_ref[...] = y

        @pl.when(c != 0)
        def _():
            y_ref[...] = y_ref[...] + y


def _mlp_tc(x_pad, w_pad, gate_up_proj, down_proj, eot, tot):
    grid_spec = pltpu.PrefetchScalarGridSpec(
        num_scalar_prefetch=2,
        grid=(NT, NKC),
        in_specs=[
            pl.BlockSpec((TILE, H),
                         lambda i, c, eot, tot:
                         (jnp.minimum(i, tot[0] - 1), 0)),
            pl.BlockSpec((1, IC, H),
                         lambda i, c, eot, tot: (eot[i], c, 0)),
            pl.BlockSpec((1, IC, H),
                         lambda i, c, eot, tot: (eot[i], NKC + c, 0)),
            pl.BlockSpec((1, H, IC),
                         lambda i, c, eot, tot: (eot[i], 0, c)),
            pl.BlockSpec((TILE, 1),
                         lambda i, c, eot, tot:
                         (jnp.minimum(i, tot[0] - 1), 0)),
        ],
        out_specs=pl.BlockSpec((TILE, H), lambda i, c, eot, tot: (i, 0)),
    )
    return pl.pallas_call(
        _mlp_body,
        grid_spec=grid_spec,
        out_shape=jax.ShapeDtypeStruct((P_PAD, H), jnp.float32),
        compiler_params=pltpu.CompilerParams(
            dimension_semantics=("arbitrary", "arbitrary")),
    )(eot, tot, x_pad, gate_up_proj, gate_up_proj, down_proj,
      w_pad.reshape(P_PAD, 1))


@functools.cache
def _combine_sc_call():
    mesh = plsc.VectorSubcoreMesh(core_axis_name="c", subcore_axis_name="s")

    @functools.partial(
        pl.kernel,
        mesh=mesh,
        out_type=jax.ShapeDtypeStruct((T, H), jnp.float32),
        scratch_types=[
            pltpu.VMEM((CHC, H), jnp.float32),
            pltpu.VMEM((CHC, H), jnp.float32),
            pltpu.VMEM((CHC,), jnp.int32),
            pltpu.VMEM((CHC,), jnp.int32),
            pltpu.SemaphoreType.DMA,
        ],
    )
    def combine(ypad_hbm, pos_e_hbm, pos_o_hbm, out_hbm,
                a_v, b_v, ie_v, io_v, sem):
        wid = lax.axis_index("s") * NC + lax.axis_index("c")
        _combine_body(ypad_hbm, pos_e_hbm, pos_o_hbm, out_hbm,
                      a_v, b_v, ie_v, io_v, sem, wid)

    return combine


def _combine_body(ypad_hbm, pos_e_hbm, pos_o_hbm, out_hbm,
                  a_v, b_v, ie_v, io_v, sem, wid):
    for j in range(TPW // CHC):
        base = wid * TPW + j * CHC
        pltpu.sync_copy(pos_e_hbm.at[wid, j], ie_v)
        pltpu.sync_copy(pos_o_hbm.at[wid, j], io_v)
        g1 = pltpu.async_copy(ypad_hbm.at[ie_v], a_v, sem)
        g2 = pltpu.async_copy(ypad_hbm.at[io_v], b_v, sem)
        g1.wait()
        g2.wait()

        def row(r, _):
            for c in range(H // 16):
                sl = pl.ds(c * 16, 16)
                a_v[r, sl] = a_v[r, sl] + b_v[r, sl]
            return _

        lax.fori_loop(0, CHC, row, None)
        pltpu.sync_copy(a_v, out_hbm.at[pl.ds(base, CHC)])


def kernel(hidden_states, top_k_index, top_k_weights, gate_up_proj, down_proj):
    pos, eot, tot = _route(top_k_index)
    pos_e3 = pos[:, 0].reshape(NW, TPW // CHD, CHD)
    pos_o3 = pos[:, 1].reshape(NW, TPW // CHD, CHD)
    w_e3 = top_k_weights[:, 0].astype(jnp.float32).reshape(NW, TPW // CHD, CHD)
    w_o3 = top_k_weights[:, 1].astype(jnp.float32).reshape(NW, TPW // CHD, CHD)

    x_pad, w_pad = _dispatch_sc_call()(hidden_states, pos_e3, pos_o3,
                                       w_e3, w_o3)
    y_pad = _mlp_tc(x_pad, w_pad, gate_up_proj, down_proj, eot, tot)

    pos_ec = pos[:, 0].reshape(NW, TPW // CHC, CHC)
    pos_oc = pos[:, 1].reshape(NW, TPW // CHC, CHC)
    return _combine_sc_call()(y_pad, pos_ec, pos_oc)


# speedup vs baseline: 1.4803x; 1.4803x over previous
import functools

import jax
import jax.numpy as jnp
from jax import lax
from jax.experimental import pallas as pl
from jax.experimental.pallas import tpu as pltpu
from jax.experimental.pallas import tpu_sc as plsc

E = 16
H = 1024
I = 768
K = 2
T = 4096
P = T * K
TILE = 256
NT = P // TILE + E
P_PAD = NT * TILE

NC, NS = 2, 16
NW = NC * NS
TPW = T // NW
CHD = 64
CHC = 32


def _route(top_k_index):
    e = top_k_index.reshape(-1).astype(jnp.int32)
    oh = (e[:, None] == jnp.arange(E, dtype=jnp.int32)[None, :]).astype(jnp.int32)
    cum = jnp.cumsum(oh, axis=0)
    counts = cum[-1]
    rank = jnp.sum(cum * oh, axis=1) - 1
    padded = ((counts + TILE - 1) // TILE) * TILE
    pad_start = jnp.concatenate([jnp.zeros(1, jnp.int32),
                                 jnp.cumsum(padded)[:-1].astype(jnp.int32)])
    pos = (pad_start[e] + rank).reshape(T, K)
    tiles_per_e = padded // TILE
    cum_tiles = jnp.cumsum(tiles_per_e)
    tot = cum_tiles[E - 1].astype(jnp.int32).reshape(1)
    tids = jnp.arange(NT, dtype=jnp.int32)
    eot = jnp.minimum(jnp.searchsorted(cum_tiles, tids, side="right"),
                      E - 1).astype(jnp.int32)
    live_tile = tids < tot[0]
    first = (live_tile & jnp.concatenate(
        [jnp.ones(1, bool), eot[1:] != eot[:-1]])).astype(jnp.int32)
    ordm = jnp.maximum(jnp.cumsum(first) - 1, 0).astype(jnp.int32)
    live_e = padded > 0
    r = jnp.cumsum(live_e.astype(jnp.int32)) - 1
    lex = jnp.zeros(E, jnp.int32).at[jnp.where(live_e, r, E)].set(
        jnp.arange(E, dtype=jnp.int32), mode="drop")
    nlive = jnp.sum(live_e.astype(jnp.int32)).reshape(1)
    return pos, eot, tot, first, ordm, lex, nlive


@functools.cache
def _dispatch_sc_call():
    mesh = plsc.VectorSubcoreMesh(core_axis_name="c", subcore_axis_name="s")

    @functools.partial(
        pl.kernel,
        mesh=mesh,
        out_type=(jax.ShapeDtypeStruct((P_PAD, H), jnp.float32),
                  jax.ShapeDtypeStruct((P_PAD,), jnp.float32)),
        scratch_types=[
            pltpu.VMEM((CHD, H), jnp.float32),
            pltpu.VMEM((CHD,), jnp.int32),
            pltpu.VMEM((CHD,), jnp.int32),
            pltpu.VMEM((CHD,), jnp.float32),
            pltpu.VMEM((CHD,), jnp.float32),
            pltpu.SemaphoreType.DMA,
        ],
    )
    def dispatch(hidden_hbm, pos_e_hbm, pos_o_hbm, w_e_hbm, w_o_hbm,
                 xpad_hbm, wpad_hbm, rows_v, ie_v, io_v, we_v, wo_v, sem):
        wid = lax.axis_index("s") * NC + lax.axis_index("c")
        _dispatch_body(hidden_hbm, pos_e_hbm, pos_o_hbm, w_e_hbm, w_o_hbm,
                       xpad_hbm, wpad_hbm, rows_v, ie_v, io_v, we_v, wo_v,
                       sem, wid)

    return dispatch


def _dispatch_body(hidden_hbm, pos_e_hbm, pos_o_hbm, w_e_hbm, w_o_hbm,
                   xpad_hbm, wpad_hbm, rows_v, ie_v, io_v, we_v, wo_v,
                   sem, wid):
    for j in range(TPW // CHD):
        base = wid * TPW + j * CHD
        pltpu.sync_copy(pos_e_hbm.at[wid, j], ie_v)
        pltpu.sync_copy(pos_o_hbm.at[wid, j], io_v)
        pltpu.sync_copy(w_e_hbm.at[wid, j], we_v)
        pltpu.sync_copy(w_o_hbm.at[wid, j], wo_v)
        pltpu.sync_copy(hidden_hbm.at[pl.ds(base, CHD)], rows_v)
        c1 = pltpu.async_copy(rows_v, xpad_hbm.at[ie_v], sem)
        c1.wait()
        c2 = pltpu.async_copy(rows_v, xpad_hbm.at[io_v], sem)
        c2.wait()
        c3 = pltpu.async_copy(we_v, wpad_hbm.at[ie_v], sem)
        c3.wait()
        c4 = pltpu.async_copy(wo_v, wpad_hbm.at[io_v], sem)
        c4.wait()


def _mlp_body(eot_ref, tot_ref, first_ref, ordm_ref, lex_ref, nlive_ref,
              x_ref, wgu_hbm, wd_hbm, wrow_ref, y_ref,
              wgu_buf, wd_buf, sem_gu, sem_d):
    i = pl.program_id(0)

    def issue(o, slot):
        e = lex_ref[o]
        pltpu.make_async_copy(wgu_hbm.at[e], wgu_buf.at[slot],
                              sem_gu.at[slot]).start()
        pltpu.make_async_copy(wd_hbm.at[e], wd_buf.at[slot],
                              sem_d.at[slot]).start()

    @pl.when(i == 0)
    def _():
        issue(0, 0)

    @pl.when(first_ref[i] == 1)
    def _():
        o = ordm_ref[i]
        slot = lax.rem(o, 2)

        @pl.when(o + 1 < nlive_ref[0])
        def _():
            issue(o + 1, lax.rem(o + 1, 2))

        e = lex_ref[o]
        pltpu.make_async_copy(wgu_hbm.at[e], wgu_buf.at[slot],
                              sem_gu.at[slot]).wait()
        pltpu.make_async_copy(wd_hbm.at[e], wd_buf.at[slot],
                              sem_d.at[slot]).wait()

    @pl.when(i < tot_ref[0])
    def _():
        slot = lax.rem(ordm_ref[i], 2)
        x = x_ref[...]
        gate = lax.dot_general(x, wgu_buf[slot, :I, :],
                               (((1,), (1,)), ((), ())),
                               preferred_element_type=jnp.float32,
                               precision=lax.Precision.DEFAULT)
        up = lax.dot_general(x, wgu_buf[slot, I:, :],
                             (((1,), (1,)), ((), ())),
                             preferred_element_type=jnp.float32,
                             precision=lax.Precision.DEFAULT)
        h = gate * jax.nn.sigmoid(gate) * up
        y = lax.dot_general(h, wd_buf[slot],
                            (((1,), (1,)), ((), ())),
                            preferred_element_type=jnp.float32,
                            precision=lax.Precision.DEFAULT)
        y_ref[...] = y * wrow_ref[...]


def _mlp_tc(x_pad, w_pad, gate_up_proj, down_proj, eot, tot,
            first, ordm, lex, nlive):
    grid_spec = pltpu.PrefetchScalarGridSpec(
        num_scalar_prefetch=6,
        grid=(NT,),
        in_specs=[
            pl.BlockSpec((TILE, H),
                         lambda i, *refs: (jnp.minimum(i, refs[1][0] - 1), 0)),
            pl.BlockSpec(memory_space=pl.ANY),
            pl.BlockSpec(memory_space=pl.ANY),
            pl.BlockSpec((TILE, 1),
                         lambda i, *refs: (jnp.minimum(i, refs[1][0] - 1), 0)),
        ],
        out_specs=pl.BlockSpec((TILE, H), lambda i, *refs: (i, 0)),
        scratch_shapes=[
            pltpu.VMEM((2, 2 * I, H), jnp.float32),
            pltpu.VMEM((2, H, I), jnp.float32),
            pltpu.SemaphoreType.DMA((2,)),
            pltpu.SemaphoreType.DMA((2,)),
        ],
    )
    return pl.pallas_call(
        _mlp_body,
        grid_spec=grid_spec,
        out_shape=jax.ShapeDtypeStruct((P_PAD, H), jnp.float32),
        compiler_params=pltpu.CompilerParams(
            dimension_semantics=("arbitrary",)),
    )(eot, tot, first, ordm, lex, nlive,
      x_pad, gate_up_proj, down_proj, w_pad.reshape(P_PAD, 1))


@functools.cache
def _combine_sc_call():
    mesh = plsc.VectorSubcoreMesh(core_axis_name="c", subcore_axis_name="s")

    @functools.partial(
        pl.kernel,
        mesh=mesh,
        out_type=jax.ShapeDtypeStruct((T, H), jnp.float32),
        scratch_types=[
            pltpu.VMEM((CHC, H), jnp.float32),
            pltpu.VMEM((CHC, H), jnp.float32),
            pltpu.VMEM((CHC,), jnp.int32),
            pltpu.VMEM((CHC,), jnp.int32),
            pltpu.SemaphoreType.DMA,
        ],
    )
    def combine(ypad_hbm, pos_e_hbm, pos_o_hbm, out_hbm,
                a_v, b_v, ie_v, io_v, sem):
        wid = lax.axis_index("s") * NC + lax.axis_index("c")
        _combine_body(ypad_hbm, pos_e_hbm, pos_o_hbm, out_hbm,
                      a_v, b_v, ie_v, io_v, sem, wid)

    return combine


def _combine_body(ypad_hbm, pos_e_hbm, pos_o_hbm, out_hbm,
                  a_v, b_v, ie_v, io_v, sem, wid):
    for j in range(TPW // CHC):
        base = wid * TPW + j * CHC
        pltpu.sync_copy(pos_e_hbm.at[wid, j], ie_v)
        pltpu.sync_copy(pos_o_hbm.at[wid, j], io_v)
        g1 = pltpu.async_copy(ypad_hbm.at[ie_v], a_v, sem)
        g2 = pltpu.async_copy(ypad_hbm.at[io_v], b_v, sem)
        g1.wait()
        g2.wait()

        def row(r, _):
            for c in range(H // 16):
                sl = pl.ds(c * 16, 16)
                a_v[r, sl] = a_v[r, sl] + b_v[r, sl]
            return _

        lax.fori_loop(0, CHC, row, None)
        pltpu.sync_copy(a_v, out_hbm.at[pl.ds(base, CHC)])


def kernel(hidden_states, top_k_index, top_k_weights, gate_up_proj, down_proj):
    pos, eot, tot, first, ordm, lex, nlive = _route(top_k_index)
    pos_e3 = pos[:, 0].reshape(NW, TPW // CHD, CHD)
    pos_o3 = pos[:, 1].reshape(NW, TPW // CHD, CHD)
    w_e3 = top_k_weights[:, 0].astype(jnp.float32).reshape(NW, TPW // CHD, CHD)
    w_o3 = top_k_weights[:, 1].astype(jnp.float32).reshape(NW, TPW // CHD, CHD)

    x_pad, w_pad = _dispatch_sc_call()(hidden_states, pos_e3, pos_o3,
                                       w_e3, w_o3)
    y_pad = _mlp_tc(x_pad, w_pad, gate_up_proj, down_proj, eot, tot,
                    first, ordm, lex, nlive)

    pos_ec = pos[:, 0].reshape(NW, TPW // CHC, CHC)
    pos_oc = pos[:, 1].reshape(NW, TPW // CHC, CHC)
    return _combine_sc_call()(y_pad, pos_ec, pos_oc)


# speedup vs baseline: 1.5130x; 1.0221x over previous
import functools

import jax
import jax.numpy as jnp
from jax import lax
from jax.experimental import pallas as pl
from jax.experimental.pallas import tpu as pltpu
from jax.experimental.pallas import tpu_sc as plsc

E = 16
H = 1024
I = 768
K = 2
T = 4096
P = T * K
TILE = 256
NT = P // TILE + E
P_PAD = NT * TILE

NC, NS = 2, 16
NW = NC * NS
TPW = T // NW
CHD = 64
CHC = 32


def _route(top_k_index):
    e = top_k_index.reshape(-1).astype(jnp.int32)
    oh_t = (e[None, :] == jnp.arange(E, dtype=jnp.int32)[:, None]
            ).astype(jnp.int32)
    cum2 = jnp.cumsum(oh_t, axis=1)
    counts = cum2[:, -1]
    rank = jnp.sum(cum2 * oh_t, axis=0) - 1
    padded = ((counts + TILE - 1) // TILE) * TILE
    pad_start = jnp.concatenate([jnp.zeros(1, jnp.int32),
                                 jnp.cumsum(padded)[:-1].astype(jnp.int32)])
    pos = (pad_start[e] + rank).reshape(T, K)
    tiles_per_e = padded // TILE
    cum_tiles = jnp.cumsum(tiles_per_e)
    tot = cum_tiles[E - 1].astype(jnp.int32).reshape(1)
    tids = jnp.arange(NT, dtype=jnp.int32)
    eot = jnp.minimum(
        jnp.sum((cum_tiles[None, :] <= tids[:, None]).astype(jnp.int32),
                axis=1), E - 1).astype(jnp.int32)
    live_tile = tids < tot[0]
    first = (live_tile & jnp.concatenate(
        [jnp.ones(1, bool), eot[1:] != eot[:-1]])).astype(jnp.int32)
    ordm = jnp.maximum(jnp.cumsum(first) - 1, 0).astype(jnp.int32)
    live_e = padded > 0
    r = jnp.cumsum(live_e.astype(jnp.int32)) - 1
    lex = jnp.zeros(E, jnp.int32).at[jnp.where(live_e, r, E)].set(
        jnp.arange(E, dtype=jnp.int32), mode="drop")
    nlive = jnp.sum(live_e.astype(jnp.int32)).reshape(1)
    return pos, eot, tot, first, ordm, lex, nlive


@functools.cache
def _dispatch_sc_call():
    mesh = plsc.VectorSubcoreMesh(core_axis_name="c", subcore_axis_name="s")

    @functools.partial(
        pl.kernel,
        mesh=mesh,
        out_type=(jax.ShapeDtypeStruct((P_PAD, H), jnp.float32),
                  jax.ShapeDtypeStruct((P_PAD,), jnp.float32)),
        scratch_types=[
            pltpu.VMEM((CHD, H), jnp.float32),
            pltpu.VMEM((CHD,), jnp.int32),
            pltpu.VMEM((CHD,), jnp.int32),
            pltpu.VMEM((CHD,), jnp.float32),
            pltpu.VMEM((CHD,), jnp.float32),
            pltpu.SemaphoreType.DMA,
        ],
    )
    def dispatch(hidden_hbm, pos_e_hbm, pos_o_hbm, w_e_hbm, w_o_hbm,
                 xpad_hbm, wpad_hbm, rows_v, ie_v, io_v, we_v, wo_v, sem):
        wid = lax.axis_index("s") * NC + lax.axis_index("c")
        _dispatch_body(hidden_hbm, pos_e_hbm, pos_o_hbm, w_e_hbm, w_o_hbm,
                       xpad_hbm, wpad_hbm, rows_v, ie_v, io_v, we_v, wo_v,
                       sem, wid)

    return dispatch


def _dispatch_body(hidden_hbm, pos_e_hbm, pos_o_hbm, w_e_hbm, w_o_hbm,
                   xpad_hbm, wpad_hbm, rows_v, ie_v, io_v, we_v, wo_v,
                   sem, wid):
    for j in range(TPW // CHD):
        base = wid * TPW + j * CHD
        pltpu.sync_copy(pos_e_hbm.at[wid, j], ie_v)
        pltpu.sync_copy(pos_o_hbm.at[wid, j], io_v)
        pltpu.sync_copy(w_e_hbm.at[wid, j], we_v)
        pltpu.sync_copy(w_o_hbm.at[wid, j], wo_v)
        pltpu.sync_copy(hidden_hbm.at[pl.ds(base, CHD)], rows_v)
        c1 = pltpu.async_copy(rows_v, xpad_hbm.at[ie_v], sem)
        c1.wait()
        c2 = pltpu.async_copy(rows_v, xpad_hbm.at[io_v], sem)
        c2.wait()
        c3 = pltpu.async_copy(we_v, wpad_hbm.at[ie_v], sem)
        c3.wait()
        c4 = pltpu.async_copy(wo_v, wpad_hbm.at[io_v], sem)
        c4.wait()


def _mlp_body(eot_ref, tot_ref, first_ref, ordm_ref, lex_ref, nlive_ref,
              x_ref, wgu_hbm, wd_hbm, wrow_ref, y_ref,
              wgu_buf, wd_buf, sem_gu, sem_d):
    i = pl.program_id(0)

    def issue(o, slot):
        e = lex_ref[o]
        pltpu.make_async_copy(wgu_hbm.at[e], wgu_buf.at[slot],
                              sem_gu.at[slot]).start()
        pltpu.make_async_copy(wd_hbm.at[e], wd_buf.at[slot],
                              sem_d.at[slot]).start()

    @pl.when(i == 0)
    def _():
        issue(0, 0)

        @pl.when(nlive_ref[0] > 1)
        def _():
            issue(1, 1)

    @pl.when(first_ref[i] == 1)
    def _():
        o = ordm_ref[i]
        slot = lax.rem(o, 3)

        @pl.when(o + 2 < nlive_ref[0])
        def _():
            issue(o + 2, lax.rem(o + 2, 3))

        e = lex_ref[o]
        pltpu.make_async_copy(wgu_hbm.at[e], wgu_buf.at[slot],
                              sem_gu.at[slot]).wait()
        pltpu.make_async_copy(wd_hbm.at[e], wd_buf.at[slot],
                              sem_d.at[slot]).wait()

    @pl.when(i < tot_ref[0])
    def _():
        slot = lax.rem(ordm_ref[i], 3)
        x = x_ref[...]
        gate = lax.dot_general(x, wgu_buf[slot, :I, :],
                               (((1,), (1,)), ((), ())),
                               preferred_element_type=jnp.float32,
                               precision=lax.Precision.DEFAULT)
        up = lax.dot_general(x, wgu_buf[slot, I:, :],
                             (((1,), (1,)), ((), ())),
                             preferred_element_type=jnp.float32,
                             precision=lax.Precision.DEFAULT)
        h = gate * jax.nn.sigmoid(gate) * up
        y = lax.dot_general(h, wd_buf[slot],
                            (((1,), (1,)), ((), ())),
                            preferred_element_type=jnp.float32,
                            precision=lax.Precision.DEFAULT)
        y_ref[...] = y * wrow_ref[...]


def _mlp_tc(x_pad, w_pad, gate_up_proj, down_proj, eot, tot,
            first, ordm, lex, nlive):
    grid_spec = pltpu.PrefetchScalarGridSpec(
        num_scalar_prefetch=6,
        grid=(NT,),
        in_specs=[
            pl.BlockSpec((TILE, H),
                         lambda i, *refs: (jnp.minimum(i, refs[1][0] - 1), 0)),
            pl.BlockSpec(memory_space=pl.ANY),
            pl.BlockSpec(memory_space=pl.ANY),
            pl.BlockSpec((TILE, 1),
                         lambda i, *refs: (jnp.minimum(i, refs[1][0] - 1), 0)),
        ],
        out_specs=pl.BlockSpec((TILE, H), lambda i, *refs: (i, 0)),
        scratch_shapes=[
            pltpu.VMEM((3, 2 * I, H), jnp.float32),
            pltpu.VMEM((3, H, I), jnp.float32),
            pltpu.SemaphoreType.DMA((3,)),
            pltpu.SemaphoreType.DMA((3,)),
        ],
    )
    return pl.pallas_call(
        _mlp_body,
        grid_spec=grid_spec,
        out_shape=jax.ShapeDtypeStruct((P_PAD, H), jnp.float32),
        compiler_params=pltpu.CompilerParams(
            dimension_semantics=("arbitrary",)),
    )(eot, tot, first, ordm, lex, nlive,
      x_pad, gate_up_proj, down_proj, w_pad.reshape(P_PAD, 1))


@functools.cache
def _combine_sc_call():
    mesh = plsc.VectorSubcoreMesh(core_axis_name="c", subcore_axis_name="s")

    @functools.partial(
        pl.kernel,
        mesh=mesh,
        out_type=jax.ShapeDtypeStruct((T, H), jnp.float32),
        scratch_types=[
            pltpu.VMEM((CHC, H), jnp.float32),
            pltpu.VMEM((CHC, H), jnp.float32),
            pltpu.VMEM((CHC,), jnp.int32),
            pltpu.VMEM((CHC,), jnp.int32),
            pltpu.SemaphoreType.DMA,
        ],
    )
    def combine(ypad_hbm, pos_e_hbm, pos_o_hbm, out_hbm,
                a_v, b_v, ie_v, io_v, sem):
        wid = lax.axis_index("s") * NC + lax.axis_index("c")
        _combine_body(ypad_hbm, pos_e_hbm, pos_o_hbm, out_hbm,
                      a_v, b_v, ie_v, io_v, sem, wid)

    return combine


def _combine_body(ypad_hbm, pos_e_hbm, pos_o_hbm, out_hbm,
                  a_v, b_v, ie_v, io_v, sem, wid):
    for j in range(TPW // CHC):
        base = wid * TPW + j * CHC
        pltpu.sync_copy(pos_e_hbm.at[wid, j], ie_v)
        pltpu.sync_copy(pos_o_hbm.at[wid, j], io_v)
        g1 = pltpu.async_copy(ypad_hbm.at[ie_v], a_v, sem)
        g2 = pltpu.async_copy(ypad_hbm.at[io_v], b_v, sem)
        g1.wait()
        g2.wait()

        def row(r, _):
            for c in range(H // 16):
                sl = pl.ds(c * 16, 16)
                a_v[r, sl] = a_v[r, sl] + b_v[r, sl]
            return _

        lax.fori_loop(0, CHC, row, None)
        pltpu.sync_copy(a_v, out_hbm.at[pl.ds(base, CHC)])


def kernel(hidden_states, top_k_index, top_k_weights, gate_up_proj, down_proj):
    pos, eot, tot, first, ordm, lex, nlive = _route(top_k_index)
    pos_e3 = pos[:, 0].reshape(NW, TPW // CHD, CHD)
    pos_o3 = pos[:, 1].reshape(NW, TPW // CHD, CHD)
    w_e3 = top_k_weights[:, 0].astype(jnp.float32).reshape(NW, TPW // CHD, CHD)
    w_o3 = top_k_weights[:, 1].astype(jnp.float32).reshape(NW, TPW // CHD, CHD)

    x_pad, w_pad = _dispatch_sc_call()(hidden_states, pos_e3, pos_o3,
                                       w_e3, w_o3)
    y_pad = _mlp_tc(x_pad, w_pad, gate_up_proj, down_proj, eot, tot,
                    first, ordm, lex, nlive)

    pos_ec = pos[:, 0].reshape(NW, TPW // CHC, CHC)
    pos_oc = pos[:, 1].reshape(NW, TPW // CHC, CHC)
    return _combine_sc_call()(y_pad, pos_ec, pos_oc)


# speedup vs baseline: 1.5960x; 1.0549x over previous
import functools

import jax
import jax.numpy as jnp
from jax import lax
from jax.experimental import pallas as pl
from jax.experimental.pallas import tpu as pltpu
from jax.experimental.pallas import tpu_sc as plsc

E = 16
H = 1024
I = 768
K = 2
T = 4096
P = T * K
TILE = 256
NT = P // TILE + E
P_PAD = NT * TILE

NC, NS = 2, 16
NW = NC * NS
TPW = T // NW
CHD = 32
NCHD = TPW // CHD
CHC = 16
NCHC = TPW // CHC


def _route_pos(top_k_index):
    e = top_k_index.reshape(-1).astype(jnp.int32)
    oh_t = (e[None, :] == jnp.arange(E, dtype=jnp.int32)[:, None]
            ).astype(jnp.int32)
    cum2 = jnp.cumsum(oh_t, axis=1)
    counts = cum2[:, -1]
    rank = jnp.sum(cum2 * oh_t, axis=0) - 1
    padded = ((counts + TILE - 1) // TILE) * TILE
    pad_start = jnp.concatenate([jnp.zeros(1, jnp.int32),
                                 jnp.cumsum(padded)[:-1].astype(jnp.int32)])
    pos = (pad_start[e] + rank).reshape(T, K)
    return pos, padded


def _route_tiles(padded):
    tiles_per_e = padded // TILE
    cum_tiles = jnp.cumsum(tiles_per_e)
    tot = cum_tiles[E - 1].astype(jnp.int32).reshape(1)
    tids = jnp.arange(NT, dtype=jnp.int32)
    eot = jnp.minimum(
        jnp.sum((cum_tiles[None, :] <= tids[:, None]).astype(jnp.int32),
                axis=1), E - 1).astype(jnp.int32)
    live_tile = tids < tot[0]
    first = (live_tile & jnp.concatenate(
        [jnp.ones(1, bool), eot[1:] != eot[:-1]])).astype(jnp.int32)
    ordm = jnp.maximum(jnp.cumsum(first) - 1, 0).astype(jnp.int32)
    live_e = padded > 0
    r = jnp.cumsum(live_e.astype(jnp.int32)) - 1
    lex = jnp.zeros(E, jnp.int32).at[jnp.where(live_e, r, E)].set(
        jnp.arange(E, dtype=jnp.int32), mode="drop")
    nlive = jnp.sum(live_e.astype(jnp.int32)).reshape(1)
    return eot, tot, first, ordm, lex, nlive


@functools.cache
def _dispatch_sc_call():
    mesh = plsc.VectorSubcoreMesh(core_axis_name="c", subcore_axis_name="s")

    @functools.partial(
        pl.kernel,
        mesh=mesh,
        out_type=(jax.ShapeDtypeStruct((P_PAD, H), jnp.float32),
                  jax.ShapeDtypeStruct((P_PAD,), jnp.float32)),
        scratch_types=[
            pltpu.VMEM((2, CHD, H), jnp.float32),
            pltpu.VMEM((NCHD, CHD), jnp.int32),
            pltpu.VMEM((NCHD, CHD), jnp.int32),
            pltpu.VMEM((NCHD, CHD), jnp.float32),
            pltpu.VMEM((NCHD, CHD), jnp.float32),
            pltpu.SemaphoreType.DMA,
            pltpu.SemaphoreType.DMA,
        ],
    )
    def dispatch(hidden_hbm, pos_e_hbm, pos_o_hbm, w_e_hbm, w_o_hbm,
                 xpad_hbm, wpad_hbm, rows_v, pe_v, po_v, we_v, wo_v,
                 sem_ld, sem_st):
        wid = lax.axis_index("s") * NC + lax.axis_index("c")
        base = wid * TPW
        pltpu.sync_copy(pos_e_hbm.at[wid], pe_v)
        pltpu.sync_copy(pos_o_hbm.at[wid], po_v)
        pltpu.sync_copy(w_e_hbm.at[wid], we_v)
        pltpu.sync_copy(w_o_hbm.at[wid], wo_v)
        loads = [None] * NCHD
        loads[0] = pltpu.async_copy(hidden_hbm.at[pl.ds(base, CHD)],
                                    rows_v.at[0], sem_ld)
        sts = [None] * NCHD
        for j in range(NCHD):
            s = j % 2
            if j >= 1:
                for c in sts[j - 1]:
                    c.wait()
            if j + 1 < NCHD:
                loads[j + 1] = pltpu.async_copy(
                    hidden_hbm.at[pl.ds(base + (j + 1) * CHD, CHD)],
                    rows_v.at[1 - s], sem_ld)
            loads[j].wait()
            sts[j] = [
                pltpu.async_copy(rows_v.at[s], xpad_hbm.at[pe_v.at[j]],
                                 sem_st),
                pltpu.async_copy(rows_v.at[s], xpad_hbm.at[po_v.at[j]],
                                 sem_st),
                pltpu.async_copy(we_v.at[j], wpad_hbm.at[pe_v.at[j]],
                                 sem_st),
                pltpu.async_copy(wo_v.at[j], wpad_hbm.at[po_v.at[j]],
                                 sem_st),
            ]
        for c in sts[NCHD - 1]:
            c.wait()

    return dispatch


def _mlp_body(eot_ref, tot_ref, first_ref, ordm_ref, lex_ref, nlive_ref,
              x_ref, wgu_hbm, wd_hbm, wrow_ref, y_ref,
              wgu_buf, wd_buf, sem_gu, sem_d):
    i = pl.program_id(0)

    def issue(o, slot):
        e = lex_ref[o]
        pltpu.make_async_copy(wgu_hbm.at[e], wgu_buf.at[slot],
                              sem_gu.at[slot]).start()
        pltpu.make_async_copy(wd_hbm.at[e], wd_buf.at[slot],
                              sem_d.at[slot]).start()

    @pl.when(i == 0)
    def _():
        issue(0, 0)

        @pl.when(nlive_ref[0] > 1)
        def _():
            issue(1, 1)

    @pl.when(first_ref[i] == 1)
    def _():
        o = ordm_ref[i]
        slot = lax.rem(o, 3)

        @pl.when(o + 2 < nlive_ref[0])
        def _():
            issue(o + 2, lax.rem(o + 2, 3))

        e = lex_ref[o]
        pltpu.make_async_copy(wgu_hbm.at[e], wgu_buf.at[slot],
                              sem_gu.at[slot]).wait()
        pltpu.make_async_copy(wd_hbm.at[e], wd_buf.at[slot],
                              sem_d.at[slot]).wait()

    @pl.when(i < tot_ref[0])
    def _():
        slot = lax.rem(ordm_ref[i], 3)
        x = x_ref[...]
        gate = lax.dot_general(x, wgu_buf[slot, :I, :],
                               (((1,), (1,)), ((), ())),
                               preferred_element_type=jnp.float32,
                               precision=lax.Precision.DEFAULT)
        up = lax.dot_general(x, wgu_buf[slot, I:, :],
                             (((1,), (1,)), ((), ())),
                             preferred_element_type=jnp.float32,
                             precision=lax.Precision.DEFAULT)
        h = gate * jax.nn.sigmoid(gate) * up
        y = lax.dot_general(h, wd_buf[slot],
                            (((1,), (1,)), ((), ())),
                            preferred_element_type=jnp.float32,
                            precision=lax.Precision.DEFAULT)
        y_ref[...] = y * wrow_ref[...]


def _mlp_tc(x_pad, w_pad, gate_up_proj, down_proj, eot, tot,
            first, ordm, lex, nlive):
    grid_spec = pltpu.PrefetchScalarGridSpec(
        num_scalar_prefetch=6,
        grid=(NT,),
        in_specs=[
            pl.BlockSpec((TILE, H),
                         lambda i, *refs: (jnp.minimum(i, refs[1][0] - 1), 0)),
            pl.BlockSpec(memory_space=pl.ANY),
            pl.BlockSpec(memory_space=pl.ANY),
            pl.BlockSpec((TILE, 1),
                         lambda i, *refs: (jnp.minimum(i, refs[1][0] - 1), 0)),
        ],
        out_specs=pl.BlockSpec((TILE, H),
                               lambda i, *refs:
                               (jnp.minimum(i, refs[1][0] - 1), 0)),
        scratch_shapes=[
            pltpu.VMEM((3, 2 * I, H), jnp.float32),
            pltpu.VMEM((3, H, I), jnp.float32),
            pltpu.SemaphoreType.DMA((3,)),
            pltpu.SemaphoreType.DMA((3,)),
        ],
    )
    return pl.pallas_call(
        _mlp_body,
        grid_spec=grid_spec,
        out_shape=jax.ShapeDtypeStruct((P_PAD, H), jnp.float32),
        compiler_params=pltpu.CompilerParams(
            dimension_semantics=("arbitrary",)),
    )(eot, tot, first, ordm, lex, nlive,
      x_pad, gate_up_proj, down_proj, w_pad.reshape(P_PAD, 1))


@functools.cache
def _combine_sc_call():
    mesh = plsc.VectorSubcoreMesh(core_axis_name="c", subcore_axis_name="s")

    @functools.partial(
        pl.kernel,
        mesh=mesh,
        out_type=jax.ShapeDtypeStruct((T, H), jnp.float32),
        scratch_types=[
            pltpu.VMEM((2, CHC, H), jnp.float32),
            pltpu.VMEM((2, CHC, H), jnp.float32),
            pltpu.VMEM((NCHC, CHC), jnp.int32),
            pltpu.VMEM((NCHC, CHC), jnp.int32),
            pltpu.SemaphoreType.DMA,
            pltpu.SemaphoreType.DMA,
        ],
    )
    def combine(ypad_hbm, pos_e_hbm, pos_o_hbm, out_hbm,
                a_v, b_v, pe_v, po_v, sem_g, sem_s):
        wid = lax.axis_index("s") * NC + lax.axis_index("c")
        base = wid * TPW
        pltpu.sync_copy(pos_e_hbm.at[wid], pe_v)
        pltpu.sync_copy(pos_o_hbm.at[wid], po_v)
        gathers = [None] * NCHC
        gathers[0] = [
            pltpu.async_copy(ypad_hbm.at[pe_v.at[0]], a_v.at[0], sem_g),
            pltpu.async_copy(ypad_hbm.at[po_v.at[0]], b_v.at[0], sem_g),
        ]
        stores = [None] * NCHC
        for j in range(NCHC):
            s = j % 2
            if j >= 1:
                stores[j - 1].wait()
            if j + 1 < NCHC:
                gathers[j + 1] = [
                    pltpu.async_copy(ypad_hbm.at[pe_v.at[j + 1]],
                                     a_v.at[1 - s], sem_g),
                    pltpu.async_copy(ypad_hbm.at[po_v.at[j + 1]],
                                     b_v.at[1 - s], sem_g),
                ]
            for g in gathers[j]:
                g.wait()

            def row(r, _, s=s):
                for c in range(H // 16):
                    sl = pl.ds(c * 16, 16)
                    a_v[s, r, sl] = a_v[s, r, sl] + b_v[s, r, sl]
                return _

            lax.fori_loop(0, CHC, row, None)
            stores[j] = pltpu.async_copy(
                a_v.at[s], out_hbm.at[pl.ds(base + j * CHC, CHC)], sem_s)
        stores[NCHC - 1].wait()

    return combine


def kernel(hidden_states, top_k_index, top_k_weights, gate_up_proj, down_proj):
    pos, padded = _route_pos(top_k_index)
    pos_e3 = pos[:, 0].reshape(NW, NCHD, CHD)
    pos_o3 = pos[:, 1].reshape(NW, NCHD, CHD)
    w_e3 = top_k_weights[:, 0].astype(jnp.float32).reshape(NW, NCHD, CHD)
    w_o3 = top_k_weights[:, 1].astype(jnp.float32).reshape(NW, NCHD, CHD)

    x_pad, w_pad = _dispatch_sc_call()(hidden_states, pos_e3, pos_o3,
                                       w_e3, w_o3)
    eot, tot, first, ordm, lex, nlive = _route_tiles(padded)
    y_pad = _mlp_tc(x_pad, w_pad, gate_up_proj, down_proj, eot, tot,
                    first, ordm, lex, nlive)

    pos_ec = pos[:, 0].reshape(NW, NCHC, CHC)
    pos_oc = pos[:, 1].reshape(NW, NCHC, CHC)
    return _combine_sc_call()(y_pad, pos_ec, pos_oc)


# speedup vs baseline: 1.6827x; 1.0543x over previous
import functools

import jax
import jax.numpy as jnp
from jax import lax
from jax.experimental import pallas as pl
from jax.experimental.pallas import tpu as pltpu
from jax.experimental.pallas import tpu_sc as plsc

E = 16
H = 1024
I = 768
K = 2
T = 4096
P = T * K
TILE = 256
NT = P // TILE + E
P_PAD = NT * TILE

NC, NS = 2, 16
NW = NC * NS
TPW = T // NW
CHD = 32
NCHD = TPW // CHD
CHC = 16
NCHC = TPW // CHC


def _route_pos(top_k_index):
    e = top_k_index.astype(jnp.int32).T.reshape(-1)
    oh_t = (e[None, :] == jnp.arange(E, dtype=jnp.int32)[:, None]
            ).astype(jnp.int32)
    cum2 = jnp.cumsum(oh_t, axis=1)
    counts = cum2[:, -1]
    rank = jnp.sum(cum2 * oh_t, axis=0) - 1
    padded = ((counts + TILE - 1) // TILE) * TILE
    pad_start = jnp.concatenate([jnp.zeros(1, jnp.int32),
                                 jnp.cumsum(padded)[:-1].astype(jnp.int32)])
    pos = pad_start[e] + rank
    return pos, padded


def _route_tiles(padded):
    tiles_per_e = padded // TILE
    cum_tiles = jnp.cumsum(tiles_per_e)
    tot = cum_tiles[E - 1].astype(jnp.int32).reshape(1)
    tids = jnp.arange(NT, dtype=jnp.int32)
    eot = jnp.minimum(
        jnp.sum((cum_tiles[None, :] <= tids[:, None]).astype(jnp.int32),
                axis=1), E - 1).astype(jnp.int32)
    live_tile = tids < tot[0]
    first = (live_tile & jnp.concatenate(
        [jnp.ones(1, bool), eot[1:] != eot[:-1]])).astype(jnp.int32)
    ordm = jnp.maximum(jnp.cumsum(first) - 1, 0).astype(jnp.int32)
    live_e = padded > 0
    r = jnp.cumsum(live_e.astype(jnp.int32)) - 1
    lex = jnp.zeros(E, jnp.int32).at[jnp.where(live_e, r, E)].set(
        jnp.arange(E, dtype=jnp.int32), mode="drop")
    nlive = jnp.sum(live_e.astype(jnp.int32)).reshape(1)
    return eot, tot, first, ordm, lex, nlive


@functools.cache
def _dispatch_sc_call():
    mesh = plsc.VectorSubcoreMesh(core_axis_name="c", subcore_axis_name="s")

    @functools.partial(
        pl.kernel,
        mesh=mesh,
        out_type=(jax.ShapeDtypeStruct((P_PAD, H), jnp.float32),
                  jax.ShapeDtypeStruct((P_PAD,), jnp.float32)),
        scratch_types=[
            pltpu.VMEM((3, CHD, H), jnp.float32),
            pltpu.VMEM((NCHD, CHD), jnp.int32),
            pltpu.VMEM((NCHD, CHD), jnp.int32),
            pltpu.VMEM((NCHD, CHD), jnp.float32),
            pltpu.VMEM((NCHD, CHD), jnp.float32),
            pltpu.SemaphoreType.DMA,
            pltpu.SemaphoreType.DMA,
        ],
    )
    def dispatch(hidden_hbm, pos_e_hbm, pos_o_hbm, w_e_hbm, w_o_hbm,
                 xpad_hbm, wpad_hbm, rows_v, pe_v, po_v, we_v, wo_v,
                 sem_ld, sem_st):
        wid = lax.axis_index("s") * NC + lax.axis_index("c")
        base = wid * TPW
        pltpu.sync_copy(pos_e_hbm.at[wid], pe_v)
        pltpu.sync_copy(pos_o_hbm.at[wid], po_v)
        pltpu.sync_copy(w_e_hbm.at[wid], we_v)
        pltpu.sync_copy(w_o_hbm.at[wid], wo_v)
        loads = [None] * NCHD
        loads[0] = pltpu.async_copy(hidden_hbm.at[pl.ds(base, CHD)],
                                    rows_v.at[0], sem_ld)
        sts = [None] * NCHD
        for j in range(NCHD):
            s = j % 3
            if j >= 2:
                for c in sts[j - 2]:
                    c.wait()
            if j + 1 < NCHD:
                loads[j + 1] = pltpu.async_copy(
                    hidden_hbm.at[pl.ds(base + (j + 1) * CHD, CHD)],
                    rows_v.at[(j + 1) % 3], sem_ld)
            loads[j].wait()
            sts[j] = [
                pltpu.async_copy(rows_v.at[s], xpad_hbm.at[pe_v.at[j]],
                                 sem_st),
                pltpu.async_copy(rows_v.at[s], xpad_hbm.at[po_v.at[j]],
                                 sem_st),
                pltpu.async_copy(we_v.at[j], wpad_hbm.at[pe_v.at[j]],
                                 sem_st),
                pltpu.async_copy(wo_v.at[j], wpad_hbm.at[po_v.at[j]],
                                 sem_st),
            ]
        for j in range(max(0, NCHD - 2), NCHD):
            for c in sts[j]:
                c.wait()

    return dispatch


def _mlp_body(eot_ref, tot_ref, first_ref, ordm_ref, lex_ref, nlive_ref,
              x_ref, wgu_hbm, wd_hbm, wrow_ref, y_ref,
              wgu_buf, wd_buf, sem_gu, sem_d):
    i = pl.program_id(0)

    def issue(o, slot):
        e = lex_ref[o]
        pltpu.make_async_copy(wgu_hbm.at[e], wgu_buf.at[slot],
                              sem_gu.at[slot]).start()
        pltpu.make_async_copy(wd_hbm.at[e], wd_buf.at[slot],
                              sem_d.at[slot]).start()

    @pl.when(i == 0)
    def _():
        issue(0, 0)

        @pl.when(nlive_ref[0] > 1)
        def _():
            issue(1, 1)

    @pl.when(first_ref[i] == 1)
    def _():
        o = ordm_ref[i]
        slot = lax.rem(o, 3)

        @pl.when(o + 2 < nlive_ref[0])
        def _():
            issue(o + 2, lax.rem(o + 2, 3))

        e = lex_ref[o]
        pltpu.make_async_copy(wgu_hbm.at[e], wgu_buf.at[slot],
                              sem_gu.at[slot]).wait()
        pltpu.make_async_copy(wd_hbm.at[e], wd_buf.at[slot],
                              sem_d.at[slot]).wait()

    @pl.when(i < tot_ref[0])
    def _():
        slot = lax.rem(ordm_ref[i], 3)
        x = x_ref[...]
        gate = lax.dot_general(x, wgu_buf[slot, :I, :],
                               (((1,), (1,)), ((), ())),
                               preferred_element_type=jnp.float32,
                               precision=lax.Precision.DEFAULT)
        up = lax.dot_general(x, wgu_buf[slot, I:, :],
                             (((1,), (1,)), ((), ())),
                             preferred_element_type=jnp.float32,
                             precision=lax.Precision.DEFAULT)
        h = gate * jax.nn.sigmoid(gate) * up
        y = lax.dot_general(h, wd_buf[slot],
                            (((1,), (1,)), ((), ())),
                            preferred_element_type=jnp.float32,
                            precision=lax.Precision.DEFAULT)
        y_ref[...] = y * wrow_ref[...]


def _mlp_tc(x_pad, w_pad, gate_up_proj, down_proj, eot, tot,
            first, ordm, lex, nlive):
    grid_spec = pltpu.PrefetchScalarGridSpec(
        num_scalar_prefetch=6,
        grid=(NT,),
        in_specs=[
            pl.BlockSpec((TILE, H),
                         lambda i, *refs: (jnp.minimum(i, refs[1][0] - 1), 0)),
            pl.BlockSpec(memory_space=pl.ANY),
            pl.BlockSpec(memory_space=pl.ANY),
            pl.BlockSpec((TILE, 1),
                         lambda i, *refs: (jnp.minimum(i, refs[1][0] - 1), 0)),
        ],
        out_specs=pl.BlockSpec((TILE, H),
                               lambda i, *refs:
                               (jnp.minimum(i, refs[1][0] - 1), 0)),
        scratch_shapes=[
            pltpu.VMEM((3, 2 * I, H), jnp.float32),
            pltpu.VMEM((3, H, I), jnp.float32),
            pltpu.SemaphoreType.DMA((3,)),
            pltpu.SemaphoreType.DMA((3,)),
        ],
    )
    return pl.pallas_call(
        _mlp_body,
        grid_spec=grid_spec,
        out_shape=jax.ShapeDtypeStruct((P_PAD, H), jnp.float32),
        compiler_params=pltpu.CompilerParams(
            dimension_semantics=("arbitrary",)),
    )(eot, tot, first, ordm, lex, nlive,
      x_pad, gate_up_proj, down_proj, w_pad.reshape(P_PAD, 1))


@functools.cache
def _combine_sc_call():
    mesh = plsc.VectorSubcoreMesh(core_axis_name="c", subcore_axis_name="s")

    @functools.partial(
        pl.kernel,
        mesh=mesh,
        out_type=jax.ShapeDtypeStruct((T, H), jnp.float32),
        scratch_types=[
            pltpu.VMEM((2, CHC, H), jnp.float32),
            pltpu.VMEM((2, CHC, H), jnp.float32),
            pltpu.VMEM((NCHC, CHC), jnp.int32),
            pltpu.VMEM((NCHC, CHC), jnp.int32),
            pltpu.SemaphoreType.DMA,
            pltpu.SemaphoreType.DMA,
        ],
    )
    def combine(ypad_hbm, pos_e_hbm, pos_o_hbm, out_hbm,
                a_v, b_v, pe_v, po_v, sem_g, sem_s):
        wid = lax.axis_index("s") * NC + lax.axis_index("c")
        base = wid * TPW
        pltpu.sync_copy(pos_e_hbm.at[wid], pe_v)
        pltpu.sync_copy(pos_o_hbm.at[wid], po_v)
        gathers = [None] * NCHC
        gathers[0] = [
            pltpu.async_copy(ypad_hbm.at[pe_v.at[0]], a_v.at[0], sem_g),
            pltpu.async_copy(ypad_hbm.at[po_v.at[0]], b_v.at[0], sem_g),
        ]
        stores = [None] * NCHC
        for j in range(NCHC):
            s = j % 2
            if j >= 1:
                stores[j - 1].wait()
            if j + 1 < NCHC:
                gathers[j + 1] = [
                    pltpu.async_copy(ypad_hbm.at[pe_v.at[j + 1]],
                                     a_v.at[1 - s], sem_g),
                    pltpu.async_copy(ypad_hbm.at[po_v.at[j + 1]],
                                     b_v.at[1 - s], sem_g),
                ]
            for g in gathers[j]:
                g.wait()

            def row(r, _, s=s):
                for c in range(H // 16):
                    sl = pl.ds(c * 16, 16)
                    a_v[s, r, sl] = a_v[s, r, sl] + b_v[s, r, sl]
                return _

            lax.fori_loop(0, CHC, row, None)
            stores[j] = pltpu.async_copy(
                a_v.at[s], out_hbm.at[pl.ds(base + j * CHC, CHC)], sem_s)
        stores[NCHC - 1].wait()

    return combine


def kernel(hidden_states, top_k_index, top_k_weights, gate_up_proj, down_proj):
    pos, padded = _route_pos(top_k_index)
    pos_e3 = pos[:T].reshape(NW, NCHD, CHD)
    pos_o3 = pos[T:].reshape(NW, NCHD, CHD)
    w_cm = top_k_weights.astype(jnp.float32).T
    w_e3 = w_cm[0].reshape(NW, NCHD, CHD)
    w_o3 = w_cm[1].reshape(NW, NCHD, CHD)

    x_pad, w_pad = _dispatch_sc_call()(hidden_states, pos_e3, pos_o3,
                                       w_e3, w_o3)
    eot, tot, first, ordm, lex, nlive = _route_tiles(padded)
    y_pad = _mlp_tc(x_pad, w_pad, gate_up_proj, down_proj, eot, tot,
                    first, ordm, lex, nlive)

    pos_ec = pos[:T].reshape(NW, NCHC, CHC)
    pos_oc = pos[T:].reshape(NW, NCHC, CHC)
    return _combine_sc_call()(y_pad, pos_ec, pos_oc)


# speedup vs baseline: 1.9684x; 1.1698x over previous
import functools

import jax
import jax.numpy as jnp
from jax import lax
from jax.experimental import pallas as pl
from jax.experimental.pallas import tpu as pltpu
from jax.experimental.pallas import tpu_sc as plsc

E = 16
H = 1024
I = 768
K = 2
T = 4096
P = T * K
TILE = 256
NT = P // TILE + E
P_PAD = NT * TILE

NC, NS = 2, 16
NW = NC * NS
TPW = T // NW
CHD = 32
NCHD = TPW // CHD
CHC = 16
NCHC = TPW // CHC


def _route_pos(top_k_index):
    e = top_k_index.astype(jnp.int32).T.reshape(-1)
    oh_t = (e[None, :] == jnp.arange(E, dtype=jnp.int32)[:, None]
            ).astype(jnp.int32)
    cum2 = jnp.cumsum(oh_t, axis=1)
    counts = cum2[:, -1]
    rank = jnp.sum(cum2 * oh_t, axis=0) - 1
    padded = ((counts + TILE - 1) // TILE) * TILE
    pad_start = jnp.concatenate([jnp.zeros(1, jnp.int32),
                                 jnp.cumsum(padded)[:-1].astype(jnp.int32)])
    pos = pad_start[e] + rank
    return pos, padded


def _route_tiles(padded):
    tiles_per_e = padded // TILE
    cum_tiles = jnp.cumsum(tiles_per_e)
    tot = cum_tiles[E - 1].astype(jnp.int32).reshape(1)
    tids = jnp.arange(NT, dtype=jnp.int32)
    eot = jnp.minimum(
        jnp.sum((cum_tiles[None, :] <= tids[:, None]).astype(jnp.int32),
                axis=1), E - 1).astype(jnp.int32)
    live_tile = tids < tot[0]
    first = (live_tile & jnp.concatenate(
        [jnp.ones(1, bool), eot[1:] != eot[:-1]])).astype(jnp.int32)
    ordm = jnp.maximum(jnp.cumsum(first) - 1, 0).astype(jnp.int32)
    live_e = padded > 0
    r = jnp.cumsum(live_e.astype(jnp.int32)) - 1
    lex = jnp.zeros(E, jnp.int32).at[jnp.where(live_e, r, E)].set(
        jnp.arange(E, dtype=jnp.int32), mode="drop")
    nlive = jnp.sum(live_e.astype(jnp.int32)).reshape(1)
    return eot, tot, first, ordm, lex, nlive


@functools.cache
def _dispatch_sc_call():
    mesh = plsc.VectorSubcoreMesh(core_axis_name="c", subcore_axis_name="s")

    @functools.partial(
        pl.kernel,
        mesh=mesh,
        out_type=jax.ShapeDtypeStruct((P_PAD, H), jnp.float32),
        scratch_types=[
            pltpu.VMEM((3, CHD, H), jnp.float32),
            pltpu.VMEM((NCHD, CHD), jnp.int32),
            pltpu.VMEM((NCHD, CHD), jnp.int32),
            pltpu.SemaphoreType.DMA,
            pltpu.SemaphoreType.DMA,
        ],
    )
    def dispatch(hidden_hbm, pos_e_hbm, pos_o_hbm,
                 xpad_hbm, rows_v, pe_v, po_v, sem_ld, sem_st):
        wid = lax.axis_index("s") * NC + lax.axis_index("c")
        base = wid * TPW
        pltpu.sync_copy(pos_e_hbm.at[wid], pe_v)
        pltpu.sync_copy(pos_o_hbm.at[wid], po_v)
        loads = [None] * NCHD
        loads[0] = pltpu.async_copy(hidden_hbm.at[pl.ds(base, CHD)],
                                    rows_v.at[0], sem_ld)
        sts = [None] * NCHD
        for j in range(NCHD):
            s = j % 3
            if j >= 2:
                for c in sts[j - 2]:
                    c.wait()
            if j + 1 < NCHD:
                loads[j + 1] = pltpu.async_copy(
                    hidden_hbm.at[pl.ds(base + (j + 1) * CHD, CHD)],
                    rows_v.at[(j + 1) % 3], sem_ld)
            loads[j].wait()
            sts[j] = [
                pltpu.async_copy(rows_v.at[s], xpad_hbm.at[pe_v.at[j]],
                                 sem_st),
                pltpu.async_copy(rows_v.at[s], xpad_hbm.at[po_v.at[j]],
                                 sem_st),
            ]
        for j in range(max(0, NCHD - 2), NCHD):
            for c in sts[j]:
                c.wait()

    return dispatch


def _mlp_body(eot_ref, tot_ref, first_ref, ordm_ref, lex_ref, nlive_ref,
              x_ref, wgu_hbm, wd_hbm, y_ref,
              wgu_buf, wd_buf, sem_gu, sem_d):
    i = pl.program_id(0)

    def issue(o, slot):
        e = lex_ref[o]
        pltpu.make_async_copy(wgu_hbm.at[e], wgu_buf.at[slot],
                              sem_gu.at[slot]).start()
        pltpu.make_async_copy(wd_hbm.at[e], wd_buf.at[slot],
                              sem_d.at[slot]).start()

    @pl.when(i == 0)
    def _():
        issue(0, 0)

        @pl.when(nlive_ref[0] > 1)
        def _():
            issue(1, 1)

    @pl.when(first_ref[i] == 1)
    def _():
        o = ordm_ref[i]
        slot = lax.rem(o, 3)

        @pl.when(o + 2 < nlive_ref[0])
        def _():
            issue(o + 2, lax.rem(o + 2, 3))

        e = lex_ref[o]
        pltpu.make_async_copy(wgu_hbm.at[e], wgu_buf.at[slot],
                              sem_gu.at[slot]).wait()
        pltpu.make_async_copy(wd_hbm.at[e], wd_buf.at[slot],
                              sem_d.at[slot]).wait()

    @pl.when(i < tot_ref[0])
    def _():
        slot = lax.rem(ordm_ref[i], 3)
        x = x_ref[...]
        gate = lax.dot_general(x, wgu_buf[slot, :I, :],
                               (((1,), (1,)), ((), ())),
                               preferred_element_type=jnp.float32,
                               precision=lax.Precision.DEFAULT)
        up = lax.dot_general(x, wgu_buf[slot, I:, :],
                             (((1,), (1,)), ((), ())),
                             preferred_element_type=jnp.float32,
                             precision=lax.Precision.DEFAULT)
        h = gate * jax.nn.sigmoid(gate) * up
        y = lax.dot_general(h, wd_buf[slot],
                            (((1,), (1,)), ((), ())),
                            preferred_element_type=jnp.float32,
                            precision=lax.Precision.DEFAULT)
        y_ref[...] = y


def _mlp_tc(x_pad, gate_up_proj, down_proj, eot, tot,
            first, ordm, lex, nlive):
    grid_spec = pltpu.PrefetchScalarGridSpec(
        num_scalar_prefetch=6,
        grid=(NT,),
        in_specs=[
            pl.BlockSpec((TILE, H),
                         lambda i, *refs: (jnp.minimum(i, refs[1][0] - 1), 0)),
            pl.BlockSpec(memory_space=pl.ANY),
            pl.BlockSpec(memory_space=pl.ANY),
        ],
        out_specs=pl.BlockSpec((TILE, H),
                               lambda i, *refs:
                               (jnp.minimum(i, refs[1][0] - 1), 0)),
        scratch_shapes=[
            pltpu.VMEM((3, 2 * I, H), jnp.float32),
            pltpu.VMEM((3, H, I), jnp.float32),
            pltpu.SemaphoreType.DMA((3,)),
            pltpu.SemaphoreType.DMA((3,)),
        ],
    )
    return pl.pallas_call(
        _mlp_body,
        grid_spec=grid_spec,
        out_shape=jax.ShapeDtypeStruct((P_PAD, H), jnp.float32),
        compiler_params=pltpu.CompilerParams(
            dimension_semantics=("arbitrary",)),
    )(eot, tot, first, ordm, lex, nlive,
      x_pad, gate_up_proj, down_proj)


@functools.cache
def _combine_sc_call():
    mesh = plsc.VectorSubcoreMesh(core_axis_name="c", subcore_axis_name="s")

    @functools.partial(
        pl.kernel,
        mesh=mesh,
        out_type=jax.ShapeDtypeStruct((T, H), jnp.float32),
        scratch_types=[
            pltpu.VMEM((2, CHC, H), jnp.float32),
            pltpu.VMEM((2, CHC, H), jnp.float32),
            pltpu.VMEM((NCHC, CHC), jnp.int32),
            pltpu.VMEM((NCHC, CHC), jnp.int32),
            pltpu.VMEM((NCHC, CHC), jnp.float32),
            pltpu.VMEM((NCHC, CHC), jnp.float32),
            pltpu.SemaphoreType.DMA,
            pltpu.SemaphoreType.DMA,
        ],
    )
    def combine(ypad_hbm, pos_e_hbm, pos_o_hbm, w_e_hbm, w_o_hbm, out_hbm,
                a_v, b_v, pe_v, po_v, we_v, wo_v, sem_g, sem_s):
        wid = lax.axis_index("s") * NC + lax.axis_index("c")
        base = wid * TPW
        pltpu.sync_copy(pos_e_hbm.at[wid], pe_v)
        pltpu.sync_copy(pos_o_hbm.at[wid], po_v)
        pltpu.sync_copy(w_e_hbm.at[wid], we_v)
        pltpu.sync_copy(w_o_hbm.at[wid], wo_v)
        gathers = [None] * NCHC
        gathers[0] = [
            pltpu.async_copy(ypad_hbm.at[pe_v.at[0]], a_v.at[0], sem_g),
            pltpu.async_copy(ypad_hbm.at[po_v.at[0]], b_v.at[0], sem_g),
        ]
        stores = [None] * NCHC
        for j in range(NCHC):
            s = j % 2
            if j >= 1:
                stores[j - 1].wait()
            if j + 1 < NCHC:
                gathers[j + 1] = [
                    pltpu.async_copy(ypad_hbm.at[pe_v.at[j + 1]],
                                     a_v.at[1 - s], sem_g),
                    pltpu.async_copy(ypad_hbm.at[po_v.at[j + 1]],
                                     b_v.at[1 - s], sem_g),
                ]
            for g in gathers[j]:
                g.wait()

            wrow0 = we_v[j]
            wrow1 = wo_v[j]

            def row(r, _, s=s, wrow0=wrow0, wrow1=wrow1):
                ir = jnp.full((16,), r, jnp.int32)
                w0 = wrow0.at[ir].get(mode="promise_in_bounds")
                w1 = wrow1.at[ir].get(mode="promise_in_bounds")
                for c in range(H // 16):
                    sl = pl.ds(c * 16, 16)
                    a_v[s, r, sl] = (a_v[s, r, sl] * w0
                                     + b_v[s, r, sl] * w1)
                return _

            lax.fori_loop(0, CHC, row, None)
            stores[j] = pltpu.async_copy(
                a_v.at[s], out_hbm.at[pl.ds(base + j * CHC, CHC)], sem_s)
        stores[NCHC - 1].wait()

    return combine


def kernel(hidden_states, top_k_index, top_k_weights, gate_up_proj, down_proj):
    pos, padded = _route_pos(top_k_index)
    pos_e3 = pos[:T].reshape(NW, NCHD, CHD)
    pos_o3 = pos[T:].reshape(NW, NCHD, CHD)

    x_pad = _dispatch_sc_call()(hidden_states, pos_e3, pos_o3)
    eot, tot, first, ordm, lex, nlive = _route_tiles(padded)
    y_pad = _mlp_tc(x_pad, gate_up_proj, down_proj, eot, tot,
                    first, ordm, lex, nlive)

    w_cm = top_k_weights.astype(jnp.float32).T
    pos_ec = pos[:T].reshape(NW, NCHC, CHC)
    pos_oc = pos[T:].reshape(NW, NCHC, CHC)
    w_ec = w_cm[0].reshape(NW, NCHC, CHC)
    w_oc = w_cm[1].reshape(NW, NCHC, CHC)
    return _combine_sc_call()(y_pad, pos_ec, pos_oc, w_ec, w_oc)
